# baseline (speedup 1.0000x reference)
import functools

import jax
import jax.numpy as jnp
from jax import lax
from jax.experimental import pallas as pl
from jax.experimental.pallas import tpu as pltpu
from jax.experimental.pallas import tpu_sc as plsc

_B = 256
_H = 128
_W = 128
_P = 32
_NW = 32
_BPW = _B // _NW
_CH = 16
_NC = _H // _CH
_CHW = _CH * _P * _W


def _sc_body(xc_hbm, yc_hbm, scal_hbm, zin_hbm, out_hbm, xv, yv, sv, buf):
    wid = lax.axis_index("s") * 2 + lax.axis_index("c")
    pltpu.sync_copy(zin_hbm, buf)
    ones = jnp.full((16,), 1.0, jnp.float32)
    zeros16 = jnp.zeros((16,), jnp.float32)
    zi16 = jnp.zeros((16,), jnp.int32)
    iota = lax.broadcasted_iota(jnp.int32, (16,), 0)
    for bi in range(_BPW):
        b = wid * _BPW + bi
        pltpu.sync_copy(xc_hbm.at[pl.ds(b, 1)], xv)
        pltpu.sync_copy(yc_hbm.at[pl.ds(b, 1)], yv)
        pltpu.sync_copy(scal_hbm.at[pl.ds(b, 1)], sv)
        rx = sv[0, 0]
        ry = sv[0, 1]
        ox = sv[0, 2]
        oy = sv[0, 3]
        cols = []
        rows = []
        ips = []
        for g in range(2):
            cols.append((xv[0, g] / rx + ox).astype(jnp.int32))
            rows.append((yv[0, g] / ry + oy).astype(jnp.int32))
            ips.append(iota + 16 * g)
        for c in range(_NC):
            fis = []
            ms = []
            for g in range(2):
                lh = rows[g] - c * _CH
                m = (lh >= 0) & (lh < _CH)
                fi = lh * (_P * _W) + ips[g] * _W + cols[g]
                fis.append(fi)
                ms.append(m)
                plsc.store_scatter(buf, [fi], ones, mask=m)
            pltpu.sync_copy(
                buf, out_hbm.at[pl.ds(b * (_H * _P * _W) + c * _CHW, _CHW)]
            )
            for g in range(2):
                plsc.store_scatter(buf, [fis[g]], zeros16, mask=ms[g])


def kernel(x, resolution, origin):
    B = x.shape[0]
    pts = x.reshape(B, _P, 2)
    xc = pts[:, :, 0].reshape(B, 2, 16)
    yc = pts[:, :, 1].reshape(B, 2, 16)
    scal = jnp.stack(
        [
            jnp.broadcast_to(resolution[:, 0:1], (B, 16)),
            jnp.broadcast_to(resolution[:, 1:2], (B, 16)),
            jnp.broadcast_to(origin[:, 0:1], (B, 16)),
            jnp.broadcast_to(origin[:, 1:2], (B, 16)),
        ],
        axis=1,
    )
    zin = jnp.zeros((_CHW,), jnp.float32)
    run = functools.partial(
        pl.kernel,
        out_type=jax.ShapeDtypeStruct((B * _H * _P * _W,), jnp.float32),
        mesh=plsc.VectorSubcoreMesh(core_axis_name="c", subcore_axis_name="s"),
        compiler_params=pltpu.CompilerParams(
            needs_layout_passes=False, use_tc_tiling_on_sc=False
        ),
        scratch_types=[
            pltpu.VMEM((1, 2, 16), jnp.float32),
            pltpu.VMEM((1, 2, 16), jnp.float32),
            pltpu.VMEM((1, 4, 16), jnp.float32),
            pltpu.VMEM((_CHW,), jnp.float32),
        ],
    )(_sc_body)
    out = run(xc, yc, scal, zin)
    return jnp.transpose(out.reshape(B, _H, _P, _W), (0, 1, 3, 2))
